# Initial kernel scaffold; baseline (speedup 1.0000x reference)
import jax
import jax.numpy as jnp
from jax.experimental import pallas as pl


def kernel(z, edge_index, emb, W0, b0, W1, b1, W2, b2, W3, b3, Wc1, bc1, Wc2, bc2, Wl1, bl1, Wl2, bl2):
    raise NotImplementedError("write your pallas kernel here")


# speedup vs baseline: 1.0301x; 1.0301x over previous
import jax
import jax.numpy as jnp
from jax import lax
from jax.experimental import pallas as pl

N = 100000
NP = 102400
BR = 4096
GRID = NP // BR
H = 32
K = 30
NEG = -3.0e38


def _k_first(h0, dout, din, W, y_o, no_o, ni_o):
    no = lax.rsqrt(jnp.maximum(dout[...], 1.0))
    ni = lax.rsqrt(jnp.maximum(din[...], 1.0))
    no_o[...] = no
    ni_o[...] = ni
    y_o[...] = jnp.dot(h0[...] * no, W[...], preferred_element_type=jnp.float32)


def _k_mid(agg, b, ni, no, W, h_o, y_o, rm_o):
    h = jnp.tanh(agg[...] * ni[...] + b[...])
    h_o[...] = h
    y_o[...] = jnp.dot(h * no[...], W[...], preferred_element_type=jnp.float32)
    rm = jnp.max(h, axis=1, keepdims=True)
    rows = pl.program_id(0) * BR + lax.broadcasted_iota(jnp.int32, (BR, 1), 0)
    rm_o[...] = jnp.where(rows < N, rm, NEG)


def _k_last(agg, b, ni, h_o, rm_o):
    h = jnp.tanh(agg[...] * ni[...] + b[...])
    h_o[...] = h
    rows = pl.program_id(0) * BR + lax.broadcasted_iota(jnp.int32, (BR, 1), 0)
    rm_o[...] = jnp.where(rows < N, h, NEG)


def _k_topk(r1, r2, r3, r4, idx_o):
    m = jnp.maximum(jnp.maximum(r1[...], r2[...]), jnp.maximum(r3[...], r4[...]))
    R = NP // 128
    flat = (lax.broadcasted_iota(jnp.int32, (R, 128), 0) * 128
            + lax.broadcasted_iota(jnp.int32, (R, 128), 1))
    lane = lax.broadcasted_iota(jnp.int32, (1, 32), 1)
    out = jnp.zeros((1, 32), jnp.int32)
    for k in range(K):
        v = jnp.max(m)
        cand = jnp.where(m >= v, flat, 2147483647)
        i = jnp.min(cand)
        out = jnp.where(lane == k, i, out)
        m = jnp.where(flat == i, NEG, m)
    idx_o[...] = out


def _k_readout(S, Wc1m, bc1, Wc2t, bc2, Wl1p, bl1, Wl2, bl2, o_ref):
    f32 = jnp.float32
    xc1 = jnp.dot(S[...], Wc1m[...], preferred_element_type=f32) + bc1[...]
    xc1 = jnp.maximum(xc1, 0.0)
    rows = [jnp.maximum(xc1[2 * p:2 * p + 1, :], xc1[2 * p + 1:2 * p + 2, :])
            for p in range(15)]
    xp = jnp.concatenate(rows, axis=0)
    W2 = Wc2t[...]
    acc = jnp.zeros((11, 32), f32)
    for d in range(5):
        acc = acc + jnp.dot(xp[d:d + 11, :], W2[d * 16:(d + 1) * 16, :],
                            preferred_element_type=f32)
    xc2 = jnp.maximum(acc + bc2[...], 0.0)
    Wl = Wl1p[...]
    o1 = bl1[...]
    for q in range(11):
        o1 = o1 + jnp.dot(xc2[q:q + 1, :], Wl[q * 32:(q + 1) * 32, :],
                          preferred_element_type=f32)
    o1 = jnp.maximum(o1, 0.0)
    o_ref[...] = jnp.dot(o1, Wl2[...], preferred_element_type=f32) + bl2[...]


def _bs(shape, const=False):
    if const:
        return pl.BlockSpec(shape, lambda i: tuple(0 for _ in shape))
    return pl.BlockSpec(shape, lambda i: (i,) + tuple(0 for _ in shape[1:]))


def kernel(z, edge_index, emb, W0, b0, W1, b1, W2, b2, W3, b3,
           Wc1, bc1, Wc2, bc2, Wl1, bl1, Wl2, bl2):
    f32 = jnp.float32
    src = edge_index[0].astype(jnp.int32)
    dst = edge_index[1].astype(jnp.int32)

    ones = jnp.ones(src.shape, f32)
    dout = jnp.zeros((N,), f32).at[src].add(ones)
    din = jnp.zeros((N,), f32).at[dst].add(ones)
    dout_p = jnp.concatenate([dout, jnp.ones((NP - N,), f32)]).reshape(NP, 1)
    din_p = jnp.concatenate([din, jnp.ones((NP - N,), f32)]).reshape(NP, 1)

    h0 = emb[z]
    h0p = jnp.concatenate([h0, jnp.zeros((NP - N, H), f32)], axis=0)

    sds = jax.ShapeDtypeStruct
    y0, no, ni = pl.pallas_call(
        _k_first, grid=(GRID,),
        in_specs=[_bs((BR, H)), _bs((BR, 1)), _bs((BR, 1)),
                  _bs((H, H), const=True)],
        out_specs=[_bs((BR, H)), _bs((BR, 1)), _bs((BR, 1))],
        out_shape=[sds((NP, H), f32), sds((NP, 1), f32), sds((NP, 1), f32)],
    )(h0p, dout_p, din_p, W0)

    def agg_of(y):
        return jnp.zeros((NP, y.shape[1]), f32).at[dst].add(y[src])

    def mid(agg, b, W):
        wc = W.shape[1]
        return pl.pallas_call(
            _k_mid, grid=(GRID,),
            in_specs=[_bs((BR, H)), _bs((1, H), const=True), _bs((BR, 1)),
                      _bs((BR, 1)), _bs((H, wc), const=True)],
            out_specs=[_bs((BR, H)), _bs((BR, wc)), _bs((BR, 1))],
            out_shape=[sds((NP, H), f32), sds((NP, wc), f32), sds((NP, 1), f32)],
        )(agg, b.reshape(1, H), ni, no, W)

    h1, y1, rm1 = mid(agg_of(y0), b0, W1)
    h2, y2, rm2 = mid(agg_of(y1), b1, W2)
    h3, y3, rm3 = mid(agg_of(y2), b2, W3)

    h4, rm4 = pl.pallas_call(
        _k_last, grid=(GRID,),
        in_specs=[_bs((BR, 1)), _bs((1, 1), const=True), _bs((BR, 1))],
        out_specs=[_bs((BR, 1)), _bs((BR, 1))],
        out_shape=[sds((NP, 1), f32), sds((NP, 1), f32)],
    )(agg_of(y3), b3.reshape(1, 1), ni)

    R = NP // 128
    idx2d = pl.pallas_call(
        _k_topk,
        in_specs=[pl.BlockSpec((R, 128), lambda: (0, 0))] * 4,
        out_specs=pl.BlockSpec((1, 32), lambda: (0, 0)),
        out_shape=sds((1, 32), jnp.int32),
    )(rm1.reshape(R, 128), rm2.reshape(R, 128),
      rm3.reshape(R, 128), rm4.reshape(R, 128))
    idx = idx2d[0, :K]

    feat30 = jnp.concatenate([h1[idx], h2[idx], h3[idx], h4[idx]], axis=1)
    S = jnp.sort(feat30, axis=-1)

    Wc1m = Wc1[:, 0, :].T
    Wc2t = jnp.transpose(Wc2, (2, 1, 0)).reshape(5 * 16, 32)
    Wl1p = Wl1.reshape(32, 11, 128).transpose(1, 0, 2).reshape(352, 128)

    out = pl.pallas_call(
        _k_readout,
        in_specs=[pl.BlockSpec(x.shape, lambda: tuple(0 for _ in x.shape))
                  for x in (S, Wc1m, bc1.reshape(1, 16), Wc2t,
                            bc2.reshape(1, 32), Wl1p, bl1.reshape(1, 128),
                            Wl2, bl2.reshape(1, 1))],
        out_specs=pl.BlockSpec((1, 1), lambda: (0, 0)),
        out_shape=sds((1, 1), f32),
    )(S, Wc1m, bc1.reshape(1, 16), Wc2t, bc2.reshape(1, 32),
      Wl1p, bl1.reshape(1, 128), Wl2, bl2.reshape(1, 1))
    return out


# speedup vs baseline: 1.0378x; 1.0074x over previous
import jax
import jax.numpy as jnp
from jax import lax
from jax.experimental import pallas as pl
from jax.experimental.pallas import tpu as pltpu
from jax.experimental.pallas import tpu_sc as plsc

N = 100000
NP = 102400
BR = 4096
GRID = NP // BR
H = 32
K = 30
NEG = -3.0e38


def _k_first(h0, dout, din, W, y_o, no_o, ni_o):
    no = lax.rsqrt(jnp.maximum(dout[...], 1.0))
    ni = lax.rsqrt(jnp.maximum(din[...], 1.0))
    no_o[...] = no
    ni_o[...] = ni
    y_o[...] = jnp.dot(h0[...] * no, W[...], preferred_element_type=jnp.float32)


def _k_mid(agg, b, ni, no, W, h_o, y_o, rm_o):
    h = jnp.tanh(agg[...] * ni[...] + b[...])
    h_o[...] = h
    y_o[...] = jnp.dot(h * no[...], W[...], preferred_element_type=jnp.float32)
    rm = jnp.max(h, axis=1, keepdims=True)
    rows = pl.program_id(0) * BR + lax.broadcasted_iota(jnp.int32, (BR, 1), 0)
    rm_o[...] = jnp.where(rows < N, rm, NEG)


def _k_last(agg, b, ni, h_o, rm_o):
    h = jnp.tanh(agg[...] * ni[...] + b[...])
    h_o[...] = h
    rows = pl.program_id(0) * BR + lax.broadcasted_iota(jnp.int32, (BR, 1), 0)
    rm_o[...] = jnp.where(rows < N, h, NEG)


def _k_topk(r1, r2, r3, r4, idx_o):
    m = jnp.maximum(jnp.maximum(r1[...], r2[...]), jnp.maximum(r3[...], r4[...]))
    R = NP // 128
    flat = (lax.broadcasted_iota(jnp.int32, (R, 128), 0) * 128
            + lax.broadcasted_iota(jnp.int32, (R, 128), 1))
    lane = lax.broadcasted_iota(jnp.int32, (1, 32), 1)
    out = jnp.zeros((1, 32), jnp.int32)
    for k in range(K):
        v = jnp.max(m)
        cand = jnp.where(m >= v, flat, 2147483647)
        i = jnp.min(cand)
        out = jnp.where(lane == k, i, out)
        m = jnp.where(flat == i, NEG, m)
    idx_o[...] = out


def _k_readout(S, Wc1m, bc1, Wc2t, bc2, Wl1p, bl1, Wl2, bl2, o_ref):
    f32 = jnp.float32
    xc1 = jnp.dot(S[...], Wc1m[...], preferred_element_type=f32) + bc1[...]
    xc1 = jnp.maximum(xc1, 0.0)
    rows = [jnp.maximum(xc1[2 * p:2 * p + 1, :], xc1[2 * p + 1:2 * p + 2, :])
            for p in range(15)]
    xp = jnp.concatenate(rows, axis=0)
    W2 = Wc2t[...]
    acc = jnp.zeros((11, 32), f32)
    for d in range(5):
        acc = acc + jnp.dot(xp[d:d + 11, :], W2[d * 16:(d + 1) * 16, :],
                            preferred_element_type=f32)
    xc2 = jnp.maximum(acc + bc2[...], 0.0)
    Wl = Wl1p[...]
    o1 = bl1[...]
    for q in range(11):
        o1 = o1 + jnp.dot(xc2[q:q + 1, :], Wl[q * 32:(q + 1) * 32, :],
                          preferred_element_type=f32)
    o1 = jnp.maximum(o1, 0.0)
    o_ref[...] = jnp.dot(o1, Wl2[...], preferred_element_type=f32) + bl2[...]


_SC_NC = 2
_SC_NW = 32
_BPW = NP // _SC_NW


def _sc_gather_body(table_hbm, idx_hbm, out_hbm, idx_v, rows_v, sem):
    wid = lax.axis_index("s") * _SC_NC + lax.axis_index("c")
    base = wid * _BPW
    pltpu.sync_copy(idx_hbm.at[pl.ds(base, _BPW)], idx_v)
    pltpu.async_copy(table_hbm.at[idx_v], rows_v, sem).wait()
    pltpu.sync_copy(rows_v, out_hbm.at[pl.ds(base, _BPW)])


def _sc_gather(table, idx):
    mesh = plsc.VectorSubcoreMesh(core_axis_name="c", subcore_axis_name="s")
    return pl.kernel(
        _sc_gather_body, mesh=mesh,
        compiler_params=pltpu.CompilerParams(use_tc_tiling_on_sc=False),
        out_type=jax.ShapeDtypeStruct((NP, H), jnp.float32),
        scratch_types=[
            pltpu.VMEM((_BPW,), jnp.int32),
            pltpu.VMEM((_BPW, H), jnp.float32),
            pltpu.SemaphoreType.DMA,
        ],
    )(table, idx)


def _bs(shape, const=False):
    if const:
        return pl.BlockSpec(shape, lambda i: tuple(0 for _ in shape))
    return pl.BlockSpec(shape, lambda i: (i,) + tuple(0 for _ in shape[1:]))


def kernel(z, edge_index, emb, W0, b0, W1, b1, W2, b2, W3, b3,
           Wc1, bc1, Wc2, bc2, Wl1, bl1, Wl2, bl2):
    f32 = jnp.float32
    src = edge_index[0].astype(jnp.int32)
    dst = edge_index[1].astype(jnp.int32)

    ones = jnp.ones(src.shape, f32)
    dout = jnp.zeros((N,), f32).at[src].add(ones)
    din = jnp.zeros((N,), f32).at[dst].add(ones)
    dout_p = jnp.concatenate([dout, jnp.ones((NP - N,), f32)]).reshape(NP, 1)
    din_p = jnp.concatenate([din, jnp.ones((NP - N,), f32)]).reshape(NP, 1)

    z_pad = jnp.concatenate(
        [z.astype(jnp.int32), jnp.zeros((NP - N,), jnp.int32)])
    h0p = _sc_gather(emb, z_pad)

    sds = jax.ShapeDtypeStruct
    y0, no, ni = pl.pallas_call(
        _k_first, grid=(GRID,),
        in_specs=[_bs((BR, H)), _bs((BR, 1)), _bs((BR, 1)),
                  _bs((H, H), const=True)],
        out_specs=[_bs((BR, H)), _bs((BR, 1)), _bs((BR, 1))],
        out_shape=[sds((NP, H), f32), sds((NP, 1), f32), sds((NP, 1), f32)],
    )(h0p, dout_p, din_p, W0)

    def agg_of(y):
        return jnp.zeros((NP, y.shape[1]), f32).at[dst].add(y[src])

    def mid(agg, b, W):
        wc = W.shape[1]
        return pl.pallas_call(
            _k_mid, grid=(GRID,),
            in_specs=[_bs((BR, H)), _bs((1, H), const=True), _bs((BR, 1)),
                      _bs((BR, 1)), _bs((H, wc), const=True)],
            out_specs=[_bs((BR, H)), _bs((BR, wc)), _bs((BR, 1))],
            out_shape=[sds((NP, H), f32), sds((NP, wc), f32), sds((NP, 1), f32)],
        )(agg, b.reshape(1, H), ni, no, W)

    h1, y1, rm1 = mid(agg_of(y0), b0, W1)
    h2, y2, rm2 = mid(agg_of(y1), b1, W2)
    h3, y3, rm3 = mid(agg_of(y2), b2, W3)

    h4, rm4 = pl.pallas_call(
        _k_last, grid=(GRID,),
        in_specs=[_bs((BR, 1)), _bs((1, 1), const=True), _bs((BR, 1))],
        out_specs=[_bs((BR, 1)), _bs((BR, 1))],
        out_shape=[sds((NP, 1), f32), sds((NP, 1), f32)],
    )(agg_of(y3), b3.reshape(1, 1), ni)

    R = NP // 128
    idx2d = pl.pallas_call(
        _k_topk,
        in_specs=[pl.BlockSpec((R, 128), lambda: (0, 0))] * 4,
        out_specs=pl.BlockSpec((1, 32), lambda: (0, 0)),
        out_shape=sds((1, 32), jnp.int32),
    )(rm1.reshape(R, 128), rm2.reshape(R, 128),
      rm3.reshape(R, 128), rm4.reshape(R, 128))
    idx = idx2d[0, :K]

    feat30 = jnp.concatenate([h1[idx], h2[idx], h3[idx], h4[idx]], axis=1)
    S = jnp.sort(feat30, axis=-1)

    Wc1m = Wc1[:, 0, :].T
    Wc2t = jnp.transpose(Wc2, (2, 1, 0)).reshape(5 * 16, 32)
    Wl1p = Wl1.reshape(32, 11, 128).transpose(1, 0, 2).reshape(352, 128)

    out = pl.pallas_call(
        _k_readout,
        in_specs=[pl.BlockSpec(x.shape, lambda: tuple(0 for _ in x.shape))
                  for x in (S, Wc1m, bc1.reshape(1, 16), Wc2t,
                            bc2.reshape(1, 32), Wl1p, bl1.reshape(1, 128),
                            Wl2, bl2.reshape(1, 1))],
        out_specs=pl.BlockSpec((1, 1), lambda: (0, 0)),
        out_shape=sds((1, 1), f32),
    )(S, Wc1m, bc1.reshape(1, 16), Wc2t, bc2.reshape(1, 32),
      Wl1p, bl1.reshape(1, 128), Wl2, bl2.reshape(1, 1))
    return out
